# baseline (speedup 1.0000x reference)
import functools

import jax
import jax.numpy as jnp
from jax import lax
from jax.experimental import pallas as pl
from jax.experimental.pallas import tpu as pltpu
from jax.experimental.pallas import tpu_sc as plsc

D = 1024
SCALE = 32.0

NC, NS, L = 2, 16, 16
NW = NC * NS

B = 16384
B_PER_W = B // NW
CB = 32
NCHUNK = B_PER_W // CB
SLICES_PER_ROW = D // L


def _sc_embed(idx_hbm, table_hbm, out_hbm, idx_v, rows_v, sem):
    wid = lax.axis_index("s") * NC + lax.axis_index("c")
    base = wid * B_PER_W
    pltpu.sync_copy(idx_hbm.at[pl.ds(base, B_PER_W)], idx_v)

    def chunk_body(g, carry):
        row0 = base + g * CB
        pltpu.async_copy(
            table_hbm.at[idx_v.at[pl.ds(g * CB, CB)]], rows_v, sem
        ).wait()

        def row_body(r, c0):
            def col_body(c, c1):
                sl = pl.ds(c * L, L)
                rows_v[r, sl] = rows_v[r, sl] * SCALE
                return c1
            return lax.fori_loop(0, SLICES_PER_ROW, col_body, c0)

        lax.fori_loop(0, CB, row_body, 0)
        pltpu.sync_copy(rows_v, out_hbm.at[pl.ds(row0, CB)])
        return carry

    lax.fori_loop(0, NCHUNK, chunk_body, 0)


@functools.partial(
    pl.kernel,
    mesh=plsc.VectorSubcoreMesh(core_axis_name="c", subcore_axis_name="s"),
    out_type=jax.ShapeDtypeStruct((B, D), jnp.float32),
    scratch_types=[
        pltpu.VMEM((B_PER_W,), jnp.int32),
        pltpu.VMEM((CB, D), jnp.float32),
        pltpu.SemaphoreType.DMA,
    ],
)
def _embed_call(idx_hbm, table_hbm, out_hbm, idx_v, rows_v, sem):
    _sc_embed(idx_hbm, table_hbm, out_hbm, idx_v, rows_v, sem)


def kernel(input_ids, weight):
    idx = input_ids.reshape(-1).astype(jnp.int32)
    out = _embed_call(idx, weight)
    return out.reshape(input_ids.shape + (D,))


# speedup vs baseline: 3.1025x; 3.1025x over previous
import functools

import jax
import jax.numpy as jnp
from jax import lax
from jax.experimental import pallas as pl
from jax.experimental.pallas import tpu as pltpu
from jax.experimental.pallas import tpu_sc as plsc

D = 1024
SCALE = 32.0

NC, NS, L = 2, 16, 16
NW = NC * NS

B = 16384
B_PER_W = B // NW
CB = 32
NCHUNK = B_PER_W // CB
SLICES_PER_ROW = D // L


def _scale_chunk(rows_v):
    def row_body(r, c0):
        for c in range(SLICES_PER_ROW):
            sl = pl.ds(c * L, L)
            rows_v[r, sl] = rows_v[r, sl] * SCALE
        return c0

    lax.fori_loop(0, CB, row_body, 0)


def _sc_embed(idx_hbm, table_hbm, out_hbm, idx_v, rows0, rows1, sg0, sg1,
              ss0, ss1):
    wid = lax.axis_index("s") * NC + lax.axis_index("c")
    base = wid * B_PER_W
    pltpu.sync_copy(idx_hbm.at[pl.ds(base, B_PER_W)], idx_v)

    bufs = (rows0, rows1)
    gsems = (sg0, sg1)
    ssems = (ss0, ss1)

    def gather(g, b):
        pltpu.async_copy(
            table_hbm.at[idx_v.at[pl.ds(g * CB, CB)]], bufs[b], gsems[b]
        )

    def gather_wait(g, b):
        pltpu.make_async_copy(
            table_hbm.at[idx_v.at[pl.ds(g * CB, CB)]], bufs[b], gsems[b]
        ).wait()

    def scatter(g, b):
        pltpu.async_copy(
            bufs[b], out_hbm.at[pl.ds(base + g * CB, CB)], ssems[b]
        )

    def scatter_wait(g, b):
        pltpu.make_async_copy(
            bufs[b], out_hbm.at[pl.ds(base + g * CB, CB)], ssems[b]
        ).wait()

    gather(0, 0)

    def chunk_pair(g0, carry):
        for bsel in range(2):
            g = g0 * 2 + bsel
            gather_wait(g, bsel)
            @pl.when(g >= 1)
            def _():
                scatter_wait(g - 1, 1 - bsel)

            @pl.when(g + 1 < NCHUNK)
            def _():
                gather(g + 1, 1 - bsel)

            _scale_chunk(bufs[bsel])
            scatter(g, bsel)
        return carry

    lax.fori_loop(0, NCHUNK // 2, chunk_pair, 0)
    scatter_wait(NCHUNK - 1, 1)


@functools.partial(
    pl.kernel,
    mesh=plsc.VectorSubcoreMesh(core_axis_name="c", subcore_axis_name="s"),
    out_type=jax.ShapeDtypeStruct((B, D), jnp.float32),
    scratch_types=[
        pltpu.VMEM((B_PER_W,), jnp.int32),
        pltpu.VMEM((CB, D), jnp.float32),
        pltpu.VMEM((CB, D), jnp.float32),
        pltpu.SemaphoreType.DMA,
        pltpu.SemaphoreType.DMA,
        pltpu.SemaphoreType.DMA,
        pltpu.SemaphoreType.DMA,
    ],
)
def _embed_call(idx_hbm, table_hbm, out_hbm, idx_v, rows0, rows1, sg0, sg1,
                ss0, ss1):
    _sc_embed(idx_hbm, table_hbm, out_hbm, idx_v, rows0, rows1, sg0, sg1,
              ss0, ss1)


def kernel(input_ids, weight):
    idx = input_ids.reshape(-1).astype(jnp.int32)
    out = _embed_call(idx, weight)
    return out.reshape(input_ids.shape + (D,))
